# Initial kernel scaffold; baseline (speedup 1.0000x reference)
import jax
import jax.numpy as jnp
from jax.experimental import pallas as pl


def kernel(rgb, weights, ray_indices, num_rays):
    raise NotImplementedError("write your pallas kernel here")


# speedup vs baseline: 1.7237x; 1.7237x over previous
import functools

import jax
import jax.numpy as jnp
from jax import lax
from jax.experimental import pallas as pl
from jax.experimental.pallas import tpu as pltpu
from jax.experimental.pallas import tpu_sc as plsc

N = 3200000
R = 50000
RPAD = 51200
NC, NS = 2, 16
NW = NC * NS
CHUNK = N // NW
B = 2000
NBLK = CHUNK // B
ZCH = RPAD // NS

CB = 1568
ROUT = NW * CB
OUTF = 3 * ROUT


def _acc_body(rgb_hbm, w_hbm, idx_hbm, out_hbm,
              idxv, wv, rgbv, vr, vg, vb, zbuf,
              acc_r, acc_g, acc_b, acc_w):
    c = lax.axis_index("c")
    s = lax.axis_index("s")
    wid = s * NC + c
    it = lax.iota(jnp.int32, 16)
    it3 = it * 3
    zero16 = jnp.zeros((16,), jnp.float32)

    def zloop(i, _):
        zbuf[pl.ds(i * 16, 16)] = zero16
        return 0
    lax.fori_loop(0, ZCH // 16, zloop, 0)
    zoff = s * ZCH
    pltpu.sync_copy(zbuf, acc_r.at[pl.ds(zoff, ZCH)])
    pltpu.sync_copy(zbuf, acc_g.at[pl.ds(zoff, ZCH)])
    pltpu.sync_copy(zbuf, acc_b.at[pl.ds(zoff, ZCH)])
    pltpu.sync_copy(zbuf, acc_w.at[pl.ds(zoff, ZCH)])
    plsc.subcore_barrier()

    chunk_base = wid * CHUNK

    def blk_loop(blk, _):
        base = chunk_base + blk * B
        pltpu.sync_copy(idx_hbm.at[pl.ds(base, B)], idxv)
        pltpu.sync_copy(w_hbm.at[pl.ds(base, B)], wv)
        pltpu.sync_copy(rgb_hbm.at[pl.ds(3 * base, 3 * B)], rgbv)

        def grp(g, _):
            off = g * 16
            v3 = it3 + off * 3
            rr = plsc.load_gather(rgbv, [v3])
            gg = plsc.load_gather(rgbv, [v3 + 1])
            bb = plsc.load_gather(rgbv, [v3 + 2])
            wg = wv[pl.ds(off, 16)]
            vr[pl.ds(off, 16)] = rr * wg
            vg[pl.ds(off, 16)] = gg * wg
            vb[pl.ds(off, 16)] = bb * wg
            return 0
        lax.fori_loop(0, B // 16, grp, 0)

        pltpu.sync_copy(vr, acc_r.at[idxv], add=True)
        pltpu.sync_copy(vg, acc_g.at[idxv], add=True)
        pltpu.sync_copy(vb, acc_b.at[idxv], add=True)
        pltpu.sync_copy(wv, acc_w.at[idxv], add=True)
        return 0
    lax.fori_loop(0, NBLK, blk_loop, 0)

    plsc.subcore_barrier()
    cb = c * 4 * RPAD
    pltpu.sync_copy(acc_r.at[pl.ds(zoff, ZCH)], out_hbm.at[pl.ds(cb + 0 * RPAD + zoff, ZCH)])
    pltpu.sync_copy(acc_g.at[pl.ds(zoff, ZCH)], out_hbm.at[pl.ds(cb + 1 * RPAD + zoff, ZCH)])
    pltpu.sync_copy(acc_b.at[pl.ds(zoff, ZCH)], out_hbm.at[pl.ds(cb + 2 * RPAD + zoff, ZCH)])
    pltpu.sync_copy(acc_w.at[pl.ds(zoff, ZCH)], out_hbm.at[pl.ds(cb + 3 * RPAD + zoff, ZCH)])


def _combine_body(part_hbm, out_hbm,
                  p0r, p0g, p0b, p0w, p1r, p1g, p1b, p1w, obuf):
    c = lax.axis_index("c")
    s = lax.axis_index("s")
    wid = s * NC + c
    it = lax.iota(jnp.int32, 16)
    it3 = it * 3
    lo = wid * CB
    pltpu.sync_copy(part_hbm.at[pl.ds(0 * RPAD + lo, CB)], p0r)
    pltpu.sync_copy(part_hbm.at[pl.ds(1 * RPAD + lo, CB)], p0g)
    pltpu.sync_copy(part_hbm.at[pl.ds(2 * RPAD + lo, CB)], p0b)
    pltpu.sync_copy(part_hbm.at[pl.ds(3 * RPAD + lo, CB)], p0w)
    pltpu.sync_copy(part_hbm.at[pl.ds(4 * RPAD + lo, CB)], p1r)
    pltpu.sync_copy(part_hbm.at[pl.ds(5 * RPAD + lo, CB)], p1g)
    pltpu.sync_copy(part_hbm.at[pl.ds(6 * RPAD + lo, CB)], p1b)
    pltpu.sync_copy(part_hbm.at[pl.ds(7 * RPAD + lo, CB)], p1w)

    def grp(g, _):
        off = g * 16
        d = pl.ds(off, 16)
        aw = p0w[d] + p1w[d]
        bg = 1.0 - aw
        orr = p0r[d] + p1r[d] + bg
        ogg = p0g[d] + p1g[d] + bg
        obb = p0b[d] + p1b[d] + bg
        pos = it3 + off * 3
        plsc.store_scatter(obuf, [pos], orr)
        plsc.store_scatter(obuf, [pos + 1], ogg)
        plsc.store_scatter(obuf, [pos + 2], obb)
        return 0
    lax.fori_loop(0, CB // 16, grp, 0)
    pltpu.sync_copy(obuf, out_hbm.at[pl.ds(3 * lo, 3 * CB)])


def kernel(rgb, weights, ray_indices, num_rays):
    del num_rays
    mesh = plsc.VectorSubcoreMesh(core_axis_name="c", subcore_axis_name="s")

    acc = functools.partial(
        pl.kernel,
        out_type=jax.ShapeDtypeStruct((NC * 4 * RPAD,), jnp.float32),
        mesh=mesh,
        scratch_types=[
            pltpu.VMEM((B,), jnp.int32),
            pltpu.VMEM((B,), jnp.float32),
            pltpu.VMEM((3 * B,), jnp.float32),
            pltpu.VMEM((B,), jnp.float32),
            pltpu.VMEM((B,), jnp.float32),
            pltpu.VMEM((B,), jnp.float32),
            pltpu.VMEM((ZCH,), jnp.float32),
            pltpu.VMEM_SHARED((RPAD,), jnp.float32),
            pltpu.VMEM_SHARED((RPAD,), jnp.float32),
            pltpu.VMEM_SHARED((RPAD,), jnp.float32),
            pltpu.VMEM_SHARED((RPAD,), jnp.float32),
        ],
        compiler_params=pltpu.CompilerParams(needs_layout_passes=False),
        name="rgb_seg_acc",
    )(_acc_body)

    comb = functools.partial(
        pl.kernel,
        out_type=jax.ShapeDtypeStruct((OUTF,), jnp.float32),
        mesh=mesh,
        scratch_types=[pltpu.VMEM((CB,), jnp.float32) for _ in range(8)]
        + [pltpu.VMEM((3 * CB,), jnp.float32)],
        compiler_params=pltpu.CompilerParams(needs_layout_passes=False),
        name="rgb_seg_combine",
    )(_combine_body)

    part = acc(rgb.reshape(-1), weights.reshape(-1), ray_indices)
    outflat = comb(part)
    return outflat[: 3 * R].reshape(R, 3)


# speedup vs baseline: 1.7481x; 1.0142x over previous
import functools

import jax
import jax.numpy as jnp
from jax import lax
from jax.experimental import pallas as pl
from jax.experimental.pallas import tpu as pltpu
from jax.experimental.pallas import tpu_sc as plsc

N = 3200000
R = 50000
RPAD = 51200
NC, NS = 2, 16
NW = NC * NS
CHUNK = N // NW
B = 2000
NBLK = CHUNK // B
ZCH = RPAD // NS
WMAX = 16384
PAGE = 256

CB = 1568
OUTF = 3 * NW * CB


def _acc_body(rgb_hbm, w_hbm, idx_hbm, out_hbm,
              idxv, wv, rgbv, zbuf, pidx, tiny, ecb,
              ring_r, ring_g, ring_b, ring_w,
              acc_r, acc_g, acc_b, acc_w):
    c = lax.axis_index("c")
    s = lax.axis_index("s")
    wid = s * NC + c
    it = lax.iota(jnp.int32, 16)
    it3 = it * 3
    itp1 = jnp.minimum(it + 1, 15)
    itm1 = jnp.maximum(it - 1, 0)
    lane0 = it == 0
    lane15 = it == 15
    zero16 = jnp.zeros((16,), jnp.float32)
    zero16i = jnp.zeros((16,), jnp.int32)

    def zloop(i, _):
        zbuf[pl.ds(i * 16, 16)] = zero16
        return 0
    lax.fori_loop(0, ZCH // 16, zloop, 0)
    zoff = s * ZCH
    pltpu.sync_copy(zbuf, acc_r.at[pl.ds(zoff, ZCH)])
    pltpu.sync_copy(zbuf, acc_g.at[pl.ds(zoff, ZCH)])
    pltpu.sync_copy(zbuf, acc_b.at[pl.ds(zoff, ZCH)])
    pltpu.sync_copy(zbuf, acc_w.at[pl.ds(zoff, ZCH)])

    def rz(i, _):
        d = pl.ds(i * 16, 16)
        ring_r[d] = zero16
        ring_g[d] = zero16
        ring_b[d] = zero16
        ring_w[d] = zero16
        return 0
    lax.fori_loop(0, WMAX // 16, rz, 0)
    plsc.subcore_barrier()

    chunk_base = wid * CHUNK

    pltpu.sync_copy(idx_hbm.at[pl.ds(chunk_base, 16)], tiny)
    base0 = (tiny[...][0] // PAGE) * PAGE

    def flush_page(b):
        def mkidx(k, _):
            pidx[pl.ds(k * 16, 16)] = it + (b + k * 16)
            return 0
        lax.fori_loop(0, PAGE // 16, mkidx, 0)
        pg0 = pl.multiple_of(lax.rem(b, WMAX), PAGE)
        d = pl.ds(pg0, PAGE)
        pltpu.sync_copy(ring_r.at[d], acc_r.at[pidx], add=True)
        pltpu.sync_copy(ring_g.at[d], acc_g.at[pidx], add=True)
        pltpu.sync_copy(ring_b.at[d], acc_b.at[pidx], add=True)
        pltpu.sync_copy(ring_w.at[d], acc_w.at[pidx], add=True)

        def pz(k, _):
            dd = pl.ds(pg0 + k * 16, 16)
            ring_r[dd] = zero16
            ring_g[dd] = zero16
            ring_b[dd] = zero16
            ring_w[dd] = zero16
            return 0
        lax.fori_loop(0, PAGE // 16, pz, 0)
        return b + PAGE

    def fold(prev, vcr, vcg, vcb, vcw):
        sl = jnp.full((16,), lax.rem(jnp.maximum(prev, 0), WMAX), jnp.int32)
        plsc.addupdate_scatter(ring_r, [sl], jnp.full((16,), jnp.sum(vcr), jnp.float32), mask=lane0)
        plsc.addupdate_scatter(ring_g, [sl], jnp.full((16,), jnp.sum(vcg), jnp.float32), mask=lane0)
        plsc.addupdate_scatter(ring_b, [sl], jnp.full((16,), jnp.sum(vcb), jnp.float32), mask=lane0)
        plsc.addupdate_scatter(ring_w, [sl], jnp.full((16,), jnp.sum(vcw), jnp.float32), mask=lane0)

    def blk_loop(blk, carry):
        base, prev, vcr, vcg, vcb, vcw = carry
        boff = chunk_base + blk * B
        pltpu.sync_copy(idx_hbm.at[pl.ds(boff, B)], idxv)
        pltpu.sync_copy(w_hbm.at[pl.ds(boff, B)], wv)
        pltpu.sync_copy(rgb_hbm.at[pl.ds(3 * boff, 3 * B)], rgbv)

        def grp(g, carry):
            base, prev, vcr, vcg, vcb, vcw = carry
            off = g * 16
            idx16 = idxv[pl.ds(off, 16)]
            i0 = idx16[0]
            i15 = idx16[15]
            v3 = it3 + off * 3
            wg = wv[pl.ds(off, 16)]
            vr = plsc.load_gather(rgbv, [v3]) * wg
            vg = plsc.load_gather(rgbv, [v3 + 1]) * wg
            vb = plsc.load_gather(rgbv, [v3 + 2]) * wg

            def fast(_):
                return base, prev, vcr + vr, vcg + vg, vcb + vb, vcw + wg

            def slow(_):
                fold(prev, vcr, vcg, vcb, vcw)

                def normal(_):
                    nb = lax.while_loop(
                        lambda b: i15 >= b + WMAX, flush_page, base)

                    def uni(_):
                        return nb, i15, vr, vg, vb, wg

                    def mixed(_):
                        slots = lax.rem(idx16, WMAX)
                        sh_n = plsc.load_gather(idxv, [off + itp1])
                        m = (idx16 != sh_n) | lane15
                        sh_p = plsc.load_gather(idxv, [off + itm1])
                        ms = idx16 != sh_p
                        A = plsc.cummax(jnp.where(ms, it, zero16i))
                        for ring, v in ((ring_r, vr), (ring_g, vg),
                                        (ring_b, vb), (ring_w, wg)):
                            cum = plsc.cumsum(v)
                            ecb[...] = cum - v
                            rs = plsc.load_gather(ecb, [A])
                            plsc.addupdate_scatter(ring, [slots], cum - rs, mask=m)
                        return nb, i15, zero16, zero16, zero16, zero16

                    return lax.cond(i0 == i15, uni, mixed, 0)

                def lanes(_):
                    slots = lax.rem(idx16, WMAX)
                    nb = base
                    for l in range(16):
                        ray = idx16[l]
                        nb = lax.while_loop(
                            lambda bb, ray=ray: ray >= bb + WMAX,
                            flush_page, nb)
                        lm = it == l
                        plsc.addupdate_scatter(ring_r, [slots], vr, mask=lm)
                        plsc.addupdate_scatter(ring_g, [slots], vg, mask=lm)
                        plsc.addupdate_scatter(ring_b, [slots], vb, mask=lm)
                        plsc.addupdate_scatter(ring_w, [slots], wg, mask=lm)
                    return nb, i15, zero16, zero16, zero16, zero16

                return lax.cond(i15 - i0 < WMAX - PAGE, normal, lanes, 0)

            return lax.cond((i0 == i15) & (i0 == prev), fast, slow, 0)

        carry = lax.fori_loop(0, B // 16, grp, (base, prev, vcr, vcg, vcb, vcw))
        return carry

    init = (base0, jnp.int32(-1), zero16, zero16, zero16, zero16)
    base, prev, vcr, vcg, vcb, vcw = lax.fori_loop(0, NBLK, blk_loop, init)

    fold(prev, vcr, vcg, vcb, vcw)
    lax.while_loop(lambda b: prev >= b, flush_page, base)

    plsc.subcore_barrier()
    cb = c * 4 * RPAD
    pltpu.sync_copy(acc_r.at[pl.ds(zoff, ZCH)], out_hbm.at[pl.ds(cb + 0 * RPAD + zoff, ZCH)])
    pltpu.sync_copy(acc_g.at[pl.ds(zoff, ZCH)], out_hbm.at[pl.ds(cb + 1 * RPAD + zoff, ZCH)])
    pltpu.sync_copy(acc_b.at[pl.ds(zoff, ZCH)], out_hbm.at[pl.ds(cb + 2 * RPAD + zoff, ZCH)])
    pltpu.sync_copy(acc_w.at[pl.ds(zoff, ZCH)], out_hbm.at[pl.ds(cb + 3 * RPAD + zoff, ZCH)])


def _combine_body(part_hbm, out_hbm,
                  p0r, p0g, p0b, p0w, p1r, p1g, p1b, p1w, obuf):
    c = lax.axis_index("c")
    s = lax.axis_index("s")
    wid = s * NC + c
    it = lax.iota(jnp.int32, 16)
    it3 = it * 3
    lo = wid * CB
    pltpu.sync_copy(part_hbm.at[pl.ds(0 * RPAD + lo, CB)], p0r)
    pltpu.sync_copy(part_hbm.at[pl.ds(1 * RPAD + lo, CB)], p0g)
    pltpu.sync_copy(part_hbm.at[pl.ds(2 * RPAD + lo, CB)], p0b)
    pltpu.sync_copy(part_hbm.at[pl.ds(3 * RPAD + lo, CB)], p0w)
    pltpu.sync_copy(part_hbm.at[pl.ds(4 * RPAD + lo, CB)], p1r)
    pltpu.sync_copy(part_hbm.at[pl.ds(5 * RPAD + lo, CB)], p1g)
    pltpu.sync_copy(part_hbm.at[pl.ds(6 * RPAD + lo, CB)], p1b)
    pltpu.sync_copy(part_hbm.at[pl.ds(7 * RPAD + lo, CB)], p1w)

    def grp(g, _):
        off = g * 16
        d = pl.ds(off, 16)
        aw = p0w[d] + p1w[d]
        bg = 1.0 - aw
        orr = p0r[d] + p1r[d] + bg
        ogg = p0g[d] + p1g[d] + bg
        obb = p0b[d] + p1b[d] + bg
        pos = it3 + off * 3
        plsc.store_scatter(obuf, [pos], orr)
        plsc.store_scatter(obuf, [pos + 1], ogg)
        plsc.store_scatter(obuf, [pos + 2], obb)
        return 0
    lax.fori_loop(0, CB // 16, grp, 0)
    pltpu.sync_copy(obuf, out_hbm.at[pl.ds(3 * lo, 3 * CB)])


def kernel(rgb, weights, ray_indices, num_rays):
    del num_rays
    mesh = plsc.VectorSubcoreMesh(core_axis_name="c", subcore_axis_name="s")

    acc = functools.partial(
        pl.kernel,
        out_type=jax.ShapeDtypeStruct((NC * 4 * RPAD,), jnp.float32),
        mesh=mesh,
        scratch_types=[
            pltpu.VMEM((B,), jnp.int32),
            pltpu.VMEM((B,), jnp.float32),
            pltpu.VMEM((3 * B,), jnp.float32),
            pltpu.VMEM((ZCH,), jnp.float32),
            pltpu.VMEM((PAGE,), jnp.int32),
            pltpu.VMEM((16,), jnp.int32),
            pltpu.VMEM((16,), jnp.float32),
            pltpu.VMEM((WMAX,), jnp.float32),
            pltpu.VMEM((WMAX,), jnp.float32),
            pltpu.VMEM((WMAX,), jnp.float32),
            pltpu.VMEM((WMAX,), jnp.float32),
            pltpu.VMEM_SHARED((RPAD,), jnp.float32),
            pltpu.VMEM_SHARED((RPAD,), jnp.float32),
            pltpu.VMEM_SHARED((RPAD,), jnp.float32),
            pltpu.VMEM_SHARED((RPAD,), jnp.float32),
        ],
        compiler_params=pltpu.CompilerParams(needs_layout_passes=False),
        name="rgb_seg_acc",
    )(_acc_body)

    comb = functools.partial(
        pl.kernel,
        out_type=jax.ShapeDtypeStruct((OUTF,), jnp.float32),
        mesh=mesh,
        scratch_types=[pltpu.VMEM((CB,), jnp.float32) for _ in range(8)]
        + [pltpu.VMEM((3 * CB,), jnp.float32)],
        compiler_params=pltpu.CompilerParams(needs_layout_passes=False),
        name="rgb_seg_combine",
    )(_combine_body)

    part = acc(rgb.reshape(-1), weights.reshape(-1), ray_indices)
    outflat = comb(part)
    return outflat[: 3 * R].reshape(R, 3)


# speedup vs baseline: 1.7485x; 1.0002x over previous
import functools

import jax
import jax.numpy as jnp
from jax import lax
from jax.experimental import pallas as pl
from jax.experimental.pallas import tpu as pltpu
from jax.experimental.pallas import tpu_sc as plsc

N = 3200000
R = 50000
RPAD = 51200
NC, NS = 2, 16
NW = NC * NS
CHUNK = N // NW
B = 2000
NBLK = CHUNK // B
ZCH = RPAD // NS
WMAX = 16384
PAGE = 256

CB = 1568
OUTF = 3 * NW * CB


def _acc_body(rgb_hbm, w_hbm, idx_hbm, out_hbm,
              idxv, wv, rgbv, zbuf, pidx, tiny, ecb,
              ring_r, ring_g, ring_b, ring_w,
              acc_r, acc_g, acc_b, acc_w):
    c = lax.axis_index("c")
    s = lax.axis_index("s")
    wid = s * NC + c
    it = lax.iota(jnp.int32, 16)
    it3 = it * 3
    itp1 = jnp.minimum(it + 1, 15)
    itm1 = jnp.maximum(it - 1, 0)
    lane0 = it == 0
    lane15 = it == 15
    zero16 = jnp.zeros((16,), jnp.float32)
    zero16i = jnp.zeros((16,), jnp.int32)

    def zloop(i, _):
        zbuf[pl.ds(i * 16, 16)] = zero16
        return 0
    lax.fori_loop(0, ZCH // 16, zloop, 0)
    zoff = s * ZCH
    pltpu.sync_copy(zbuf, acc_r.at[pl.ds(zoff, ZCH)])
    pltpu.sync_copy(zbuf, acc_g.at[pl.ds(zoff, ZCH)])
    pltpu.sync_copy(zbuf, acc_b.at[pl.ds(zoff, ZCH)])
    pltpu.sync_copy(zbuf, acc_w.at[pl.ds(zoff, ZCH)])

    def rz(i, _):
        d = pl.ds(i * 16, 16)
        ring_r[d] = zero16
        ring_g[d] = zero16
        ring_b[d] = zero16
        ring_w[d] = zero16
        return 0
    lax.fori_loop(0, WMAX // 16, rz, 0)
    plsc.subcore_barrier()

    chunk_base = wid * CHUNK

    pltpu.sync_copy(idx_hbm.at[pl.ds(chunk_base, 16)], tiny)
    base0 = (tiny[...][0] // PAGE) * PAGE

    def flush_page(b):
        def mkidx(k, _):
            pidx[pl.ds(k * 16, 16)] = it + (b + k * 16)
            return 0
        lax.fori_loop(0, PAGE // 16, mkidx, 0)
        pg0 = pl.multiple_of(lax.rem(b, WMAX), PAGE)
        d = pl.ds(pg0, PAGE)
        pltpu.sync_copy(ring_r.at[d], acc_r.at[pidx], add=True)
        pltpu.sync_copy(ring_g.at[d], acc_g.at[pidx], add=True)
        pltpu.sync_copy(ring_b.at[d], acc_b.at[pidx], add=True)
        pltpu.sync_copy(ring_w.at[d], acc_w.at[pidx], add=True)

        def pz(k, _):
            dd = pl.ds(pg0 + k * 16, 16)
            ring_r[dd] = zero16
            ring_g[dd] = zero16
            ring_b[dd] = zero16
            ring_w[dd] = zero16
            return 0
        lax.fori_loop(0, PAGE // 16, pz, 0)
        return b + PAGE

    def fold(prev, vcr, vcg, vcb, vcw):
        sl = jnp.full((16,), lax.rem(jnp.maximum(prev, 0), WMAX), jnp.int32)
        plsc.addupdate_scatter(ring_r, [sl], jnp.full((16,), jnp.sum(vcr), jnp.float32), mask=lane0)
        plsc.addupdate_scatter(ring_g, [sl], jnp.full((16,), jnp.sum(vcg), jnp.float32), mask=lane0)
        plsc.addupdate_scatter(ring_b, [sl], jnp.full((16,), jnp.sum(vcb), jnp.float32), mask=lane0)
        plsc.addupdate_scatter(ring_w, [sl], jnp.full((16,), jnp.sum(vcw), jnp.float32), mask=lane0)

    def blk_loop(blk, carry):
        base, prev, vcr, vcg, vcb, vcw = carry
        boff = chunk_base + blk * B
        pltpu.sync_copy(idx_hbm.at[pl.ds(boff, B)], idxv)
        pltpu.sync_copy(w_hbm.at[pl.ds(boff, B)], wv)
        pltpu.sync_copy(rgb_hbm.at[pl.ds(3 * boff, 3 * B)], rgbv)

        def grp(g, carry):
            base, prev, vcr, vcg, vcb, vcw = carry
            off = g * 16
            idx16 = idxv[pl.ds(off, 16)]
            i0 = idx16[0]
            i15 = idx16[15]
            v3 = it3 + off * 3
            wg = wv[pl.ds(off, 16)]
            vr = plsc.load_gather(rgbv, [v3]) * wg
            vg = plsc.load_gather(rgbv, [v3 + 1]) * wg
            vb = plsc.load_gather(rgbv, [v3 + 2]) * wg

            def fast(_):
                return base, prev, vcr + vr, vcg + vg, vcb + vb, vcw + wg

            def slow(_):
                fold(prev, vcr, vcg, vcb, vcw)

                def normal(_):
                    nb = lax.while_loop(
                        lambda b: i15 >= b + WMAX, flush_page, base)

                    def uni(_):
                        return nb, i15, vr, vg, vb, wg

                    def mixed(_):
                        slots = lax.rem(idx16, WMAX)
                        sh_n = plsc.load_gather(idxv, [off + itp1])
                        m = (idx16 != sh_n) | lane15
                        sh_p = plsc.load_gather(idxv, [off + itm1])
                        ms = idx16 != sh_p
                        A = plsc.cummax(jnp.where(ms, it, zero16i))
                        for ring, v in ((ring_r, vr), (ring_g, vg),
                                        (ring_b, vb), (ring_w, wg)):
                            cum = plsc.cumsum(v)
                            ecb[...] = cum - v
                            rs = plsc.load_gather(ecb, [A])
                            plsc.addupdate_scatter(ring, [slots], cum - rs, mask=m)
                        return nb, i15, zero16, zero16, zero16, zero16

                    return lax.cond(i0 == i15, uni, mixed, 0)

                def lanes(_):
                    slots = lax.rem(idx16, WMAX)
                    nb = base
                    for l in range(16):
                        ray = idx16[l]
                        nb = lax.while_loop(
                            lambda bb, ray=ray: ray >= bb + WMAX,
                            flush_page, nb)
                        lm = it == l
                        plsc.addupdate_scatter(ring_r, [slots], vr, mask=lm)
                        plsc.addupdate_scatter(ring_g, [slots], vg, mask=lm)
                        plsc.addupdate_scatter(ring_b, [slots], vb, mask=lm)
                        plsc.addupdate_scatter(ring_w, [slots], wg, mask=lm)
                    return nb, i15, zero16, zero16, zero16, zero16

                return lax.cond(i15 - i0 < WMAX - PAGE, normal, lanes, 0)

            return lax.cond((i0 == i15) & (i0 == prev), fast, slow, 0)

        carry = lax.fori_loop(0, B // 16, grp, (base, prev, vcr, vcg, vcb, vcw))
        return carry

    init = (base0, jnp.int32(-1), zero16, zero16, zero16, zero16)
    base, prev, vcr, vcg, vcb, vcw = lax.fori_loop(0, NBLK, blk_loop, init)

    fold(prev, vcr, vcg, vcb, vcw)
    lax.while_loop(lambda b: prev >= b, flush_page, base)

    plsc.subcore_barrier()
    cb = c * 4 * RPAD
    pltpu.sync_copy(acc_r.at[pl.ds(zoff, ZCH)], out_hbm.at[pl.ds(cb + 0 * RPAD + zoff, ZCH)])
    pltpu.sync_copy(acc_g.at[pl.ds(zoff, ZCH)], out_hbm.at[pl.ds(cb + 1 * RPAD + zoff, ZCH)])
    pltpu.sync_copy(acc_b.at[pl.ds(zoff, ZCH)], out_hbm.at[pl.ds(cb + 2 * RPAD + zoff, ZCH)])
    pltpu.sync_copy(acc_w.at[pl.ds(zoff, ZCH)], out_hbm.at[pl.ds(cb + 3 * RPAD + zoff, ZCH)])


def _combine_body(part_hbm, out_hbm,
                  p0r, p0g, p0b, p0w, p1r, p1g, p1b, p1w, obuf):
    c = lax.axis_index("c")
    s = lax.axis_index("s")
    wid = s * NC + c
    it = lax.iota(jnp.int32, 16)
    it3 = it * 3
    lo = wid * CB
    pltpu.sync_copy(part_hbm.at[pl.ds(0 * RPAD + lo, CB)], p0r)
    pltpu.sync_copy(part_hbm.at[pl.ds(1 * RPAD + lo, CB)], p0g)
    pltpu.sync_copy(part_hbm.at[pl.ds(2 * RPAD + lo, CB)], p0b)
    pltpu.sync_copy(part_hbm.at[pl.ds(3 * RPAD + lo, CB)], p0w)
    pltpu.sync_copy(part_hbm.at[pl.ds(4 * RPAD + lo, CB)], p1r)
    pltpu.sync_copy(part_hbm.at[pl.ds(5 * RPAD + lo, CB)], p1g)
    pltpu.sync_copy(part_hbm.at[pl.ds(6 * RPAD + lo, CB)], p1b)
    pltpu.sync_copy(part_hbm.at[pl.ds(7 * RPAD + lo, CB)], p1w)

    def grp(g, _):
        off = g * 16
        d = pl.ds(off, 16)
        aw = p0w[d] + p1w[d]
        bg = 1.0 - aw
        orr = p0r[d] + p1r[d] + bg
        ogg = p0g[d] + p1g[d] + bg
        obb = p0b[d] + p1b[d] + bg
        pos = it3 + off * 3
        plsc.store_scatter(obuf, [pos], orr)
        plsc.store_scatter(obuf, [pos + 1], ogg)
        plsc.store_scatter(obuf, [pos + 2], obb)
        return 0
    lax.fori_loop(0, CB // 16, grp, 0)
    pltpu.sync_copy(obuf, out_hbm.at[pl.ds(3 * lo, 3 * CB)])


def kernel(rgb, weights, ray_indices, num_rays):
    del num_rays
    mesh = plsc.VectorSubcoreMesh(core_axis_name="c", subcore_axis_name="s")

    acc = functools.partial(
        pl.kernel,
        out_type=jax.ShapeDtypeStruct((NC * 4 * RPAD,), jnp.float32),
        mesh=mesh,
        scratch_types=[
            pltpu.VMEM((B,), jnp.int32),
            pltpu.VMEM((B,), jnp.float32),
            pltpu.VMEM((3 * B,), jnp.float32),
            pltpu.VMEM((ZCH,), jnp.float32),
            pltpu.VMEM((PAGE,), jnp.int32),
            pltpu.VMEM((16,), jnp.int32),
            pltpu.VMEM((16,), jnp.float32),
            pltpu.VMEM((WMAX,), jnp.float32),
            pltpu.VMEM((WMAX,), jnp.float32),
            pltpu.VMEM((WMAX,), jnp.float32),
            pltpu.VMEM((WMAX,), jnp.float32),
            pltpu.VMEM_SHARED((RPAD,), jnp.float32),
            pltpu.VMEM_SHARED((RPAD,), jnp.float32),
            pltpu.VMEM_SHARED((RPAD,), jnp.float32),
            pltpu.VMEM_SHARED((RPAD,), jnp.float32),
        ],
        compiler_params=pltpu.CompilerParams(
            needs_layout_passes=False, use_tc_tiling_on_sc=True),
        name="rgb_seg_acc",
    )(_acc_body)

    comb = functools.partial(
        pl.kernel,
        out_type=jax.ShapeDtypeStruct((OUTF,), jnp.float32),
        mesh=mesh,
        scratch_types=[pltpu.VMEM((CB,), jnp.float32) for _ in range(8)]
        + [pltpu.VMEM((3 * CB,), jnp.float32)],
        compiler_params=pltpu.CompilerParams(needs_layout_passes=False),
        name="rgb_seg_combine",
    )(_combine_body)

    part = acc(rgb.reshape(-1), weights.reshape(-1), ray_indices)
    outflat = comb(part)
    return outflat[: 3 * R].reshape(R, 3)


# speedup vs baseline: 44.4914x; 25.4460x over previous
import functools

import jax
import jax.numpy as jnp
from jax import lax
from jax.experimental import pallas as pl
from jax.experimental.pallas import tpu as pltpu
from jax.experimental.pallas import tpu_sc as plsc

N = 3200000
R = 50000
RPAD = 51200
NC, NS = 2, 16
NW = NC * NS
CHUNK = 100096
B = 2048
TAIL = CHUNK - 48 * B
TAILL = N - 31 * CHUNK - 47 * B
ZCH = RPAD // NS
WMAX = 16384
PAGE = 256

CB = 1568
OUTF = 3 * NW * CB


def _acc_body(rgb_hbm, w_hbm, idx_hbm, out_hbm,
              idxv, wv, rgbv, zbuf, pidx, tiny, ecb,
              ring_r, ring_g, ring_b, ring_w,
              acc_r, acc_g, acc_b, acc_w):
    c = lax.axis_index("c")
    s = lax.axis_index("s")
    wid = s * NC + c
    it = lax.iota(jnp.int32, 16)
    itp1 = jnp.minimum(it + 1, 15)
    itm1 = jnp.maximum(it - 1, 0)
    lane0 = it == 0
    lane15 = it == 15
    zero16 = jnp.zeros((16,), jnp.float32)
    zero16i = jnp.zeros((16,), jnp.int32)
    one16i = jnp.full((16,), 1, jnp.int32)
    two16i = jnp.full((16,), 2, jnp.int32)

    def zloop(i, _):
        zbuf[pl.ds(i * 16, 16)] = zero16
        return 0
    lax.fori_loop(0, ZCH // 16, zloop, 0)
    zoff = s * ZCH
    pltpu.sync_copy(zbuf, acc_r.at[pl.ds(zoff, ZCH)])
    pltpu.sync_copy(zbuf, acc_g.at[pl.ds(zoff, ZCH)])
    pltpu.sync_copy(zbuf, acc_b.at[pl.ds(zoff, ZCH)])
    pltpu.sync_copy(zbuf, acc_w.at[pl.ds(zoff, ZCH)])

    def rz(i, _):
        d = pl.ds(i * 16, 16)
        ring_r[d] = zero16
        ring_g[d] = zero16
        ring_b[d] = zero16
        ring_w[d] = zero16
        return 0
    lax.fori_loop(0, WMAX // 16, rz, 0)
    plsc.subcore_barrier()

    chunk_base = wid * CHUNK

    pltpu.sync_copy(idx_hbm.at[pl.ds(chunk_base, 16)], tiny)
    base0 = (tiny[...][0] // PAGE) * PAGE

    def flush_page(b):
        def mkidx(k, _):
            pidx[pl.ds(k * 16, 16)] = it + (b + k * 16)
            return 0
        lax.fori_loop(0, PAGE // 16, mkidx, 0)
        pg0 = pl.multiple_of(lax.rem(b, WMAX), PAGE)
        d = pl.ds(pg0, PAGE)
        pltpu.sync_copy(ring_r.at[d], acc_r.at[pidx], add=True)
        pltpu.sync_copy(ring_g.at[d], acc_g.at[pidx], add=True)
        pltpu.sync_copy(ring_b.at[d], acc_b.at[pidx], add=True)
        pltpu.sync_copy(ring_w.at[d], acc_w.at[pidx], add=True)

        def pz(k, _):
            dd = pl.ds(pg0 + k * 16, 16)
            ring_r[dd] = zero16
            ring_g[dd] = zero16
            ring_b[dd] = zero16
            ring_w[dd] = zero16
            return 0
        lax.fori_loop(0, PAGE // 16, pz, 0)
        return b + PAGE

    def fold(prev, vcr, vcg, vcb, vcw):
        sl = jnp.full((16,), lax.rem(jnp.maximum(prev, 0), WMAX), jnp.int32)
        plsc.addupdate_scatter(ring_r, [sl], jnp.full((16,), jnp.sum(vcr), jnp.float32), mask=lane0)
        plsc.addupdate_scatter(ring_g, [sl], jnp.full((16,), jnp.sum(vcg), jnp.float32), mask=lane0)
        plsc.addupdate_scatter(ring_b, [sl], jnp.full((16,), jnp.sum(vcb), jnp.float32), mask=lane0)
        plsc.addupdate_scatter(ring_w, [sl], jnp.full((16,), jnp.sum(vcw), jnp.float32), mask=lane0)

    def process_block(boff, nsamp, carry):
        pltpu.sync_copy(idx_hbm.at[pl.ds(boff, nsamp)], idxv.at[pl.ds(0, nsamp)])
        pltpu.sync_copy(w_hbm.at[pl.ds(boff, nsamp)], wv.at[pl.ds(0, nsamp)])
        pltpu.sync_copy(rgb_hbm.at[:, pl.ds(boff, nsamp)],
                        rgbv.at[:, pl.ds(0, nsamp)])

        def grp(g, carry):
            base, prev, vcr, vcg, vcb, vcw = carry
            off = g * 16
            idx16 = idxv[pl.ds(off, 16)]
            i0 = idx16[0]
            i15 = idx16[15]
            d16 = pl.ds(off, 16)
            wg = wv[d16]
            vr = rgbv[0, d16] * wg
            vg = rgbv[1, d16] * wg
            vb = rgbv[2, d16] * wg

            def fast(_):
                return base, prev, vcr + vr, vcg + vg, vcb + vb, vcw + wg

            def slow(_):
                fold(prev, vcr, vcg, vcb, vcw)

                def normal(_):
                    nb = lax.while_loop(
                        lambda b: i15 >= b + WMAX, flush_page, base)

                    def uni(_):
                        return nb, i15, vr, vg, vb, wg

                    def mixed(_):
                        slots = lax.rem(idx16, WMAX)
                        sh_n = plsc.load_gather(idxv, [off + itp1])
                        m = (idx16 != sh_n) | lane15
                        sh_p = plsc.load_gather(idxv, [off + itm1])
                        ms = idx16 != sh_p
                        A = plsc.cummax(jnp.where(ms, it, zero16i))
                        for ring, v in ((ring_r, vr), (ring_g, vg),
                                        (ring_b, vb), (ring_w, wg)):
                            cum = plsc.cumsum(v)
                            ecb[...] = cum - v
                            rs = plsc.load_gather(ecb, [A])
                            plsc.addupdate_scatter(ring, [slots], cum - rs, mask=m)
                        return nb, i15, zero16, zero16, zero16, zero16

                    return lax.cond(i0 == i15, uni, mixed, 0)

                def lanes(_):
                    slots = lax.rem(idx16, WMAX)
                    nb = base
                    for l in range(16):
                        ray = idx16[l]
                        nb = lax.while_loop(
                            lambda bb, ray=ray: ray >= bb + WMAX,
                            flush_page, nb)
                        lm = it == l
                        plsc.addupdate_scatter(ring_r, [slots], vr, mask=lm)
                        plsc.addupdate_scatter(ring_g, [slots], vg, mask=lm)
                        plsc.addupdate_scatter(ring_b, [slots], vb, mask=lm)
                        plsc.addupdate_scatter(ring_w, [slots], wg, mask=lm)
                    return nb, i15, zero16, zero16, zero16, zero16

                return lax.cond(i15 - i0 < WMAX - PAGE, normal, lanes, 0)

            return lax.cond((i0 == i15) & (i0 == prev), fast, slow, 0)

        return lax.fori_loop(0, nsamp // 16, grp, carry)

    init = (base0, jnp.int32(-1), zero16, zero16, zero16, zero16)
    last = wid == NW - 1
    nblk = jnp.where(last, 47, 48)

    def blk_loop(blk, carry):
        return process_block(chunk_base + blk * B, B, carry)
    carry = lax.fori_loop(0, nblk, blk_loop, init)

    def tail_main(carry):
        return process_block(chunk_base + 48 * B, TAIL, carry)

    def tail_last(carry):
        return process_block(chunk_base + 47 * B, TAILL, carry)
    base, prev, vcr, vcg, vcb, vcw = lax.cond(last, tail_last, tail_main, carry)

    fold(prev, vcr, vcg, vcb, vcw)
    lax.while_loop(lambda b: prev >= b, flush_page, base)

    plsc.subcore_barrier()
    cb = c * 4 * RPAD
    pltpu.sync_copy(acc_r.at[pl.ds(zoff, ZCH)], out_hbm.at[pl.ds(cb + 0 * RPAD + zoff, ZCH)])
    pltpu.sync_copy(acc_g.at[pl.ds(zoff, ZCH)], out_hbm.at[pl.ds(cb + 1 * RPAD + zoff, ZCH)])
    pltpu.sync_copy(acc_b.at[pl.ds(zoff, ZCH)], out_hbm.at[pl.ds(cb + 2 * RPAD + zoff, ZCH)])
    pltpu.sync_copy(acc_w.at[pl.ds(zoff, ZCH)], out_hbm.at[pl.ds(cb + 3 * RPAD + zoff, ZCH)])


def _combine_body(part_hbm, out_hbm,
                  p0r, p0g, p0b, p0w, p1r, p1g, p1b, p1w, obuf):
    c = lax.axis_index("c")
    s = lax.axis_index("s")
    wid = s * NC + c
    it = lax.iota(jnp.int32, 16)
    it3 = it * 3
    bufs = (p0r, p0g, p0b, p0w, p1r, p1g, p1b, p1w)
    lo = wid * CB
    for k, b in enumerate(bufs):
        pltpu.sync_copy(part_hbm.at[pl.ds(k * RPAD + lo, CB)], b)

    def grp(g, _):
        off = g * 16
        d = pl.ds(off, 16)
        aw = p0w[d] + p1w[d]
        bg = 1.0 - aw
        orr = p0r[d] + p1r[d] + bg
        ogg = p0g[d] + p1g[d] + bg
        obb = p0b[d] + p1b[d] + bg
        pos = it3 + off * 3
        plsc.store_scatter(obuf, [pos], orr)
        plsc.store_scatter(obuf, [pos + 1], ogg)
        plsc.store_scatter(obuf, [pos + 2], obb)
        return 0
    lax.fori_loop(0, CB // 16, grp, 0)
    pltpu.sync_copy(obuf, out_hbm.at[pl.ds(3 * lo, 3 * CB)])


def kernel(rgb, weights, ray_indices, num_rays):
    del num_rays
    mesh = plsc.VectorSubcoreMesh(core_axis_name="c", subcore_axis_name="s")

    acc = functools.partial(
        pl.kernel,
        out_type=jax.ShapeDtypeStruct((NC * 4 * RPAD,), jnp.float32),
        mesh=mesh,
        scratch_types=[
            pltpu.VMEM((B,), jnp.int32),
            pltpu.VMEM((B,), jnp.float32),
            pltpu.VMEM((3, B), jnp.float32),
            pltpu.VMEM((ZCH,), jnp.float32),
            pltpu.VMEM((PAGE,), jnp.int32),
            pltpu.VMEM((16,), jnp.int32),
            pltpu.VMEM((16,), jnp.float32),
            pltpu.VMEM((WMAX,), jnp.float32),
            pltpu.VMEM((WMAX,), jnp.float32),
            pltpu.VMEM((WMAX,), jnp.float32),
            pltpu.VMEM((WMAX,), jnp.float32),
            pltpu.VMEM_SHARED((RPAD,), jnp.float32),
            pltpu.VMEM_SHARED((RPAD,), jnp.float32),
            pltpu.VMEM_SHARED((RPAD,), jnp.float32),
            pltpu.VMEM_SHARED((RPAD,), jnp.float32),
        ],
        compiler_params=pltpu.CompilerParams(
            needs_layout_passes=False, use_tc_tiling_on_sc=True),
        name="rgb_seg_acc",
    )(_acc_body)

    comb = functools.partial(
        pl.kernel,
        out_type=jax.ShapeDtypeStruct((OUTF,), jnp.float32),
        mesh=mesh,
        scratch_types=[pltpu.VMEM((CB,), jnp.float32) for _ in range(8)]
        + [pltpu.VMEM((3 * CB,), jnp.float32)],
        compiler_params=pltpu.CompilerParams(needs_layout_passes=False),
        name="rgb_seg_combine",
    )(_combine_body)

    part = acc(rgb.T, weights.reshape(-1), ray_indices)
    outflat = comb(part)
    return outflat[: 3 * R].reshape(R, 3)


# speedup vs baseline: 51.1865x; 1.1505x over previous
import functools

import jax
import jax.numpy as jnp
from jax import lax
from jax.experimental import pallas as pl
from jax.experimental.pallas import tpu as pltpu
from jax.experimental.pallas import tpu_sc as plsc

N = 3200000
R = 50000
RPAD = 51200
NC, NS = 2, 16
NW = NC * NS
CHUNK = 100096
B = 2048
TAIL = CHUNK - 48 * B
TAILL = N - 31 * CHUNK - 47 * B
ZCH = RPAD // NS
WMAX = 16384
PAGE = 256

CB = 1568
OUTF = 3 * NW * CB


def _acc_body(rgb_hbm, w_hbm, idx_hbm, out_hbm,
              idxv, wv, rgbv, zbuf, pidx, tiny, ecb, dsem,
              ring_r, ring_g, ring_b, ring_w,
              acc_r, acc_g, acc_b, acc_w):
    c = lax.axis_index("c")
    s = lax.axis_index("s")
    wid = s * NC + c
    it = lax.iota(jnp.int32, 16)
    itp1 = jnp.minimum(it + 1, 15)
    itm1 = jnp.maximum(it - 1, 0)
    lane0 = it == 0
    lane15 = it == 15
    zero16 = jnp.zeros((16,), jnp.float32)
    zero16i = jnp.zeros((16,), jnp.int32)
    one16i = jnp.full((16,), 1, jnp.int32)
    two16i = jnp.full((16,), 2, jnp.int32)

    def zloop(i, _):
        zbuf[pl.ds(i * 16, 16)] = zero16
        return 0
    lax.fori_loop(0, ZCH // 16, zloop, 0)
    zoff = s * ZCH
    pltpu.sync_copy(zbuf, acc_r.at[pl.ds(zoff, ZCH)])
    pltpu.sync_copy(zbuf, acc_g.at[pl.ds(zoff, ZCH)])
    pltpu.sync_copy(zbuf, acc_b.at[pl.ds(zoff, ZCH)])
    pltpu.sync_copy(zbuf, acc_w.at[pl.ds(zoff, ZCH)])

    def rz(i, _):
        d = pl.ds(i * 16, 16)
        ring_r[d] = zero16
        ring_g[d] = zero16
        ring_b[d] = zero16
        ring_w[d] = zero16
        return 0
    lax.fori_loop(0, WMAX // 16, rz, 0)
    plsc.subcore_barrier()

    chunk_base = wid * CHUNK

    pltpu.sync_copy(idx_hbm.at[pl.ds(chunk_base, 16)], tiny)
    base0 = (tiny[...][0] // PAGE) * PAGE

    def flush_page(b):
        def mkidx(k, _):
            pidx[pl.ds(k * 16, 16)] = it + (b + k * 16)
            return 0
        lax.fori_loop(0, PAGE // 16, mkidx, 0)
        pg0 = pl.multiple_of(lax.rem(b, WMAX), PAGE)
        d = pl.ds(pg0, PAGE)
        pltpu.sync_copy(ring_r.at[d], acc_r.at[pidx], add=True)
        pltpu.sync_copy(ring_g.at[d], acc_g.at[pidx], add=True)
        pltpu.sync_copy(ring_b.at[d], acc_b.at[pidx], add=True)
        pltpu.sync_copy(ring_w.at[d], acc_w.at[pidx], add=True)

        def pz(k, _):
            dd = pl.ds(pg0 + k * 16, 16)
            ring_r[dd] = zero16
            ring_g[dd] = zero16
            ring_b[dd] = zero16
            ring_w[dd] = zero16
            return 0
        lax.fori_loop(0, PAGE // 16, pz, 0)
        return b + PAGE

    def fold(prev, vcr, vcg, vcb, vcw):
        sl = jnp.full((16,), lax.rem(jnp.maximum(prev, 0), WMAX), jnp.int32)
        plsc.addupdate_scatter(ring_r, [sl], jnp.full((16,), jnp.sum(vcr), jnp.float32), mask=lane0)
        plsc.addupdate_scatter(ring_g, [sl], jnp.full((16,), jnp.sum(vcg), jnp.float32), mask=lane0)
        plsc.addupdate_scatter(ring_b, [sl], jnp.full((16,), jnp.sum(vcb), jnp.float32), mask=lane0)
        plsc.addupdate_scatter(ring_w, [sl], jnp.full((16,), jnp.sum(vcw), jnp.float32), mask=lane0)

    def process_block(boff, nsamp, carry):
        d1 = pltpu.async_copy(idx_hbm.at[pl.ds(boff, nsamp)],
                              idxv.at[pl.ds(0, nsamp)], dsem)
        d2 = pltpu.async_copy(w_hbm.at[pl.ds(boff, nsamp)],
                              wv.at[pl.ds(0, nsamp)], dsem)
        d3 = pltpu.async_copy(rgb_hbm.at[:, pl.ds(boff, nsamp)],
                              rgbv.at[:, pl.ds(0, nsamp)], dsem)
        d1.wait()
        d2.wait()
        d3.wait()

        def grp(g, carry):
            base, prev, vcr, vcg, vcb, vcw = carry
            off = g * 16
            idx16 = idxv[pl.ds(off, 16)]
            i0 = idx16[0]
            i15 = idx16[15]
            d16 = pl.ds(off, 16)
            wg = wv[d16]
            vr = rgbv[0, d16] * wg
            vg = rgbv[1, d16] * wg
            vb = rgbv[2, d16] * wg

            def fast(_):
                return base, prev, vcr + vr, vcg + vg, vcb + vb, vcw + wg

            def slow(_):
                fold(prev, vcr, vcg, vcb, vcw)

                def normal(_):
                    nb = lax.while_loop(
                        lambda b: i15 >= b + WMAX, flush_page, base)

                    def uni(_):
                        return nb, i15, vr, vg, vb, wg

                    def mixed(_):
                        slots = lax.rem(idx16, WMAX)
                        sh_n = plsc.load_gather(idxv, [off + itp1])
                        m = (idx16 != sh_n) | lane15
                        sh_p = plsc.load_gather(idxv, [off + itm1])
                        ms = idx16 != sh_p
                        A = plsc.cummax(jnp.where(ms, it, zero16i))
                        for ring, v in ((ring_r, vr), (ring_g, vg),
                                        (ring_b, vb), (ring_w, wg)):
                            cum = plsc.cumsum(v)
                            ecb[...] = cum - v
                            rs = plsc.load_gather(ecb, [A])
                            plsc.addupdate_scatter(ring, [slots], cum - rs, mask=m)
                        return nb, i15, zero16, zero16, zero16, zero16

                    return lax.cond(i0 == i15, uni, mixed, 0)

                def lanes(_):
                    slots = lax.rem(idx16, WMAX)
                    nb = base
                    for l in range(16):
                        ray = idx16[l]
                        nb = lax.while_loop(
                            lambda bb, ray=ray: ray >= bb + WMAX,
                            flush_page, nb)
                        lm = it == l
                        plsc.addupdate_scatter(ring_r, [slots], vr, mask=lm)
                        plsc.addupdate_scatter(ring_g, [slots], vg, mask=lm)
                        plsc.addupdate_scatter(ring_b, [slots], vb, mask=lm)
                        plsc.addupdate_scatter(ring_w, [slots], wg, mask=lm)
                    return nb, i15, zero16, zero16, zero16, zero16

                return lax.cond(i15 - i0 < WMAX - PAGE, normal, lanes, 0)

            return lax.cond((i0 == i15) & (i0 == prev), fast, slow, 0)

        return lax.fori_loop(0, nsamp // 16, grp, carry)

    init = (base0, jnp.int32(-1), zero16, zero16, zero16, zero16)
    last = wid == NW - 1
    nblk = jnp.where(last, 47, 48)

    def blk_loop(blk, carry):
        return process_block(chunk_base + blk * B, B, carry)
    carry = lax.fori_loop(0, nblk, blk_loop, init)

    def tail_main(carry):
        return process_block(chunk_base + 48 * B, TAIL, carry)

    def tail_last(carry):
        return process_block(chunk_base + 47 * B, TAILL, carry)
    base, prev, vcr, vcg, vcb, vcw = lax.cond(last, tail_last, tail_main, carry)

    fold(prev, vcr, vcg, vcb, vcw)
    lax.while_loop(lambda b: prev >= b, flush_page, base)

    plsc.subcore_barrier()
    cb = c * 4 * RPAD
    pltpu.sync_copy(acc_r.at[pl.ds(zoff, ZCH)], out_hbm.at[pl.ds(cb + 0 * RPAD + zoff, ZCH)])
    pltpu.sync_copy(acc_g.at[pl.ds(zoff, ZCH)], out_hbm.at[pl.ds(cb + 1 * RPAD + zoff, ZCH)])
    pltpu.sync_copy(acc_b.at[pl.ds(zoff, ZCH)], out_hbm.at[pl.ds(cb + 2 * RPAD + zoff, ZCH)])
    pltpu.sync_copy(acc_w.at[pl.ds(zoff, ZCH)], out_hbm.at[pl.ds(cb + 3 * RPAD + zoff, ZCH)])


def _combine_body(part_hbm, out_hbm,
                  p0r, p0g, p0b, p0w, p1r, p1g, p1b, p1w, obuf):
    c = lax.axis_index("c")
    s = lax.axis_index("s")
    wid = s * NC + c
    it = lax.iota(jnp.int32, 16)
    it3 = it * 3
    bufs = (p0r, p0g, p0b, p0w, p1r, p1g, p1b, p1w)
    lo = wid * CB
    for k, b in enumerate(bufs):
        pltpu.sync_copy(part_hbm.at[pl.ds(k * RPAD + lo, CB)], b)

    def grp(g, _):
        off = g * 16
        d = pl.ds(off, 16)
        aw = p0w[d] + p1w[d]
        bg = 1.0 - aw
        orr = p0r[d] + p1r[d] + bg
        ogg = p0g[d] + p1g[d] + bg
        obb = p0b[d] + p1b[d] + bg
        pos = it3 + off * 3
        plsc.store_scatter(obuf, [pos], orr)
        plsc.store_scatter(obuf, [pos + 1], ogg)
        plsc.store_scatter(obuf, [pos + 2], obb)
        return 0
    lax.fori_loop(0, CB // 16, grp, 0)
    pltpu.sync_copy(obuf, out_hbm.at[pl.ds(3 * lo, 3 * CB)])


def kernel(rgb, weights, ray_indices, num_rays):
    del num_rays
    mesh = plsc.VectorSubcoreMesh(core_axis_name="c", subcore_axis_name="s")

    acc = functools.partial(
        pl.kernel,
        out_type=jax.ShapeDtypeStruct((NC * 4 * RPAD,), jnp.float32),
        mesh=mesh,
        scratch_types=[
            pltpu.VMEM((B,), jnp.int32),
            pltpu.VMEM((B,), jnp.float32),
            pltpu.VMEM((3, B), jnp.float32),
            pltpu.VMEM((ZCH,), jnp.float32),
            pltpu.VMEM((PAGE,), jnp.int32),
            pltpu.VMEM((16,), jnp.int32),
            pltpu.VMEM((16,), jnp.float32),
            pltpu.SemaphoreType.DMA,
            pltpu.VMEM((WMAX,), jnp.float32),
            pltpu.VMEM((WMAX,), jnp.float32),
            pltpu.VMEM((WMAX,), jnp.float32),
            pltpu.VMEM((WMAX,), jnp.float32),
            pltpu.VMEM_SHARED((RPAD,), jnp.float32),
            pltpu.VMEM_SHARED((RPAD,), jnp.float32),
            pltpu.VMEM_SHARED((RPAD,), jnp.float32),
            pltpu.VMEM_SHARED((RPAD,), jnp.float32),
        ],
        compiler_params=pltpu.CompilerParams(
            needs_layout_passes=False, use_tc_tiling_on_sc=True),
        name="rgb_seg_acc",
    )(_acc_body)

    comb = functools.partial(
        pl.kernel,
        out_type=jax.ShapeDtypeStruct((OUTF,), jnp.float32),
        mesh=mesh,
        scratch_types=[pltpu.VMEM((CB,), jnp.float32) for _ in range(8)]
        + [pltpu.VMEM((3 * CB,), jnp.float32)],
        compiler_params=pltpu.CompilerParams(needs_layout_passes=False),
        name="rgb_seg_combine",
    )(_combine_body)

    part = acc(rgb.T, weights.reshape(-1), ray_indices)
    outflat = comb(part)
    return outflat[: 3 * R].reshape(R, 3)


# speedup vs baseline: 53.3599x; 1.0425x over previous
import functools

import jax
import jax.numpy as jnp
from jax import lax
from jax.experimental import pallas as pl
from jax.experimental.pallas import tpu as pltpu
from jax.experimental.pallas import tpu_sc as plsc

N = 3200000
R = 50000
RPAD = 51200
NC, NS = 2, 16
NW = NC * NS
CHUNK = 100096
B = 4096
TAIL = CHUNK - 24 * B
TAILL = N - 31 * CHUNK - 23 * B
ZCH = RPAD // NS
WMAX = 16384
PAGE = 256

CB = 1568
OUTF = 3 * NW * CB


def _acc_body(rgb_hbm, w_hbm, idx_hbm, out_hbm,
              idxv, wv, rgbv, zbuf, pidx, tiny, ecb, dsem,
              ring_r, ring_g, ring_b, ring_w,
              acc_r, acc_g, acc_b, acc_w):
    c = lax.axis_index("c")
    s = lax.axis_index("s")
    wid = s * NC + c
    it = lax.iota(jnp.int32, 16)
    itp1 = jnp.minimum(it + 1, 15)
    itm1 = jnp.maximum(it - 1, 0)
    lane0 = it == 0
    lane15 = it == 15
    zero16 = jnp.zeros((16,), jnp.float32)
    zero16i = jnp.zeros((16,), jnp.int32)
    one16i = jnp.full((16,), 1, jnp.int32)
    two16i = jnp.full((16,), 2, jnp.int32)

    def zloop(i, _):
        zbuf[pl.ds(i * 16, 16)] = zero16
        return 0
    lax.fori_loop(0, ZCH // 16, zloop, 0)
    zoff = s * ZCH
    pltpu.sync_copy(zbuf, acc_r.at[pl.ds(zoff, ZCH)])
    pltpu.sync_copy(zbuf, acc_g.at[pl.ds(zoff, ZCH)])
    pltpu.sync_copy(zbuf, acc_b.at[pl.ds(zoff, ZCH)])
    pltpu.sync_copy(zbuf, acc_w.at[pl.ds(zoff, ZCH)])

    def rz(i, _):
        d = pl.ds(i * 16, 16)
        ring_r[d] = zero16
        ring_g[d] = zero16
        ring_b[d] = zero16
        ring_w[d] = zero16
        return 0
    lax.fori_loop(0, WMAX // 16, rz, 0)
    plsc.subcore_barrier()

    chunk_base = wid * CHUNK

    pltpu.sync_copy(idx_hbm.at[pl.ds(chunk_base, 16)], tiny)
    base0 = (tiny[...][0] // PAGE) * PAGE

    def flush_page(b):
        def mkidx(k, _):
            pidx[pl.ds(k * 16, 16)] = it + (b + k * 16)
            return 0
        lax.fori_loop(0, PAGE // 16, mkidx, 0)
        pg0 = pl.multiple_of(lax.rem(b, WMAX), PAGE)
        d = pl.ds(pg0, PAGE)
        pltpu.sync_copy(ring_r.at[d], acc_r.at[pidx], add=True)
        pltpu.sync_copy(ring_g.at[d], acc_g.at[pidx], add=True)
        pltpu.sync_copy(ring_b.at[d], acc_b.at[pidx], add=True)
        pltpu.sync_copy(ring_w.at[d], acc_w.at[pidx], add=True)

        def pz(k, _):
            dd = pl.ds(pg0 + k * 16, 16)
            ring_r[dd] = zero16
            ring_g[dd] = zero16
            ring_b[dd] = zero16
            ring_w[dd] = zero16
            return 0
        lax.fori_loop(0, PAGE // 16, pz, 0)
        return b + PAGE

    def fold(prev, vcr, vcg, vcb, vcw):
        sl = jnp.full((16,), lax.rem(jnp.maximum(prev, 0), WMAX), jnp.int32)
        plsc.addupdate_scatter(ring_r, [sl], jnp.full((16,), jnp.sum(vcr), jnp.float32), mask=lane0)
        plsc.addupdate_scatter(ring_g, [sl], jnp.full((16,), jnp.sum(vcg), jnp.float32), mask=lane0)
        plsc.addupdate_scatter(ring_b, [sl], jnp.full((16,), jnp.sum(vcb), jnp.float32), mask=lane0)
        plsc.addupdate_scatter(ring_w, [sl], jnp.full((16,), jnp.sum(vcw), jnp.float32), mask=lane0)

    def process_block(boff, nsamp, carry):
        d1 = pltpu.async_copy(idx_hbm.at[pl.ds(boff, nsamp)],
                              idxv.at[pl.ds(0, nsamp)], dsem)
        d2 = pltpu.async_copy(w_hbm.at[pl.ds(boff, nsamp)],
                              wv.at[pl.ds(0, nsamp)], dsem)
        d3 = pltpu.async_copy(rgb_hbm.at[:, pl.ds(boff, nsamp)],
                              rgbv.at[:, pl.ds(0, nsamp)], dsem)
        d1.wait()
        d2.wait()
        d3.wait()

        def grp(g, carry):
            base, prev, vcr, vcg, vcb, vcw = carry
            off = g * 16
            idx16 = idxv[pl.ds(off, 16)]
            i0 = idx16[0]
            i15 = idx16[15]
            d16 = pl.ds(off, 16)
            wg = wv[d16]
            vr = rgbv[0, d16] * wg
            vg = rgbv[1, d16] * wg
            vb = rgbv[2, d16] * wg

            def fast(_):
                return base, prev, vcr + vr, vcg + vg, vcb + vb, vcw + wg

            def slow(_):
                fold(prev, vcr, vcg, vcb, vcw)

                def normal(_):
                    nb = lax.while_loop(
                        lambda b: i15 >= b + WMAX, flush_page, base)

                    def uni(_):
                        return nb, i15, vr, vg, vb, wg

                    def mixed(_):
                        slots = lax.rem(idx16, WMAX)
                        sh_n = plsc.load_gather(idxv, [off + itp1])
                        m = (idx16 != sh_n) | lane15
                        sh_p = plsc.load_gather(idxv, [off + itm1])
                        ms = idx16 != sh_p
                        A = plsc.cummax(jnp.where(ms, it, zero16i))
                        for ring, v in ((ring_r, vr), (ring_g, vg),
                                        (ring_b, vb), (ring_w, wg)):
                            cum = plsc.cumsum(v)
                            ecb[...] = cum - v
                            rs = plsc.load_gather(ecb, [A])
                            plsc.addupdate_scatter(ring, [slots], cum - rs, mask=m)
                        return nb, i15, zero16, zero16, zero16, zero16

                    return lax.cond(i0 == i15, uni, mixed, 0)

                def lanes(_):
                    slots = lax.rem(idx16, WMAX)
                    nb = base
                    for l in range(16):
                        ray = idx16[l]
                        nb = lax.while_loop(
                            lambda bb, ray=ray: ray >= bb + WMAX,
                            flush_page, nb)
                        lm = it == l
                        plsc.addupdate_scatter(ring_r, [slots], vr, mask=lm)
                        plsc.addupdate_scatter(ring_g, [slots], vg, mask=lm)
                        plsc.addupdate_scatter(ring_b, [slots], vb, mask=lm)
                        plsc.addupdate_scatter(ring_w, [slots], wg, mask=lm)
                    return nb, i15, zero16, zero16, zero16, zero16

                return lax.cond(i15 - i0 < WMAX - PAGE, normal, lanes, 0)

            return lax.cond((i0 == i15) & (i0 == prev), fast, slow, 0)

        return lax.fori_loop(0, nsamp // 16, grp, carry)

    init = (base0, jnp.int32(-1), zero16, zero16, zero16, zero16)
    last = wid == NW - 1
    nblk = jnp.where(last, 23, 24)

    def blk_loop(blk, carry):
        return process_block(chunk_base + blk * B, B, carry)
    carry = lax.fori_loop(0, nblk, blk_loop, init)

    def tail_main(carry):
        return process_block(chunk_base + 24 * B, TAIL, carry)

    def tail_last(carry):
        return process_block(chunk_base + 23 * B, TAILL, carry)
    base, prev, vcr, vcg, vcb, vcw = lax.cond(last, tail_last, tail_main, carry)

    fold(prev, vcr, vcg, vcb, vcw)
    lax.while_loop(lambda b: prev >= b, flush_page, base)

    plsc.subcore_barrier()
    cb = c * 4 * RPAD
    pltpu.sync_copy(acc_r.at[pl.ds(zoff, ZCH)], out_hbm.at[pl.ds(cb + 0 * RPAD + zoff, ZCH)])
    pltpu.sync_copy(acc_g.at[pl.ds(zoff, ZCH)], out_hbm.at[pl.ds(cb + 1 * RPAD + zoff, ZCH)])
    pltpu.sync_copy(acc_b.at[pl.ds(zoff, ZCH)], out_hbm.at[pl.ds(cb + 2 * RPAD + zoff, ZCH)])
    pltpu.sync_copy(acc_w.at[pl.ds(zoff, ZCH)], out_hbm.at[pl.ds(cb + 3 * RPAD + zoff, ZCH)])


def _combine_body(part_hbm, out_hbm,
                  p0r, p0g, p0b, p0w, p1r, p1g, p1b, p1w, obuf):
    c = lax.axis_index("c")
    s = lax.axis_index("s")
    wid = s * NC + c
    it = lax.iota(jnp.int32, 16)
    it3 = it * 3
    bufs = (p0r, p0g, p0b, p0w, p1r, p1g, p1b, p1w)
    lo = wid * CB
    for k, b in enumerate(bufs):
        pltpu.sync_copy(part_hbm.at[pl.ds(k * RPAD + lo, CB)], b)

    def grp(g, _):
        off = g * 16
        d = pl.ds(off, 16)
        aw = p0w[d] + p1w[d]
        bg = 1.0 - aw
        orr = p0r[d] + p1r[d] + bg
        ogg = p0g[d] + p1g[d] + bg
        obb = p0b[d] + p1b[d] + bg
        pos = it3 + off * 3
        plsc.store_scatter(obuf, [pos], orr)
        plsc.store_scatter(obuf, [pos + 1], ogg)
        plsc.store_scatter(obuf, [pos + 2], obb)
        return 0
    lax.fori_loop(0, CB // 16, grp, 0)
    pltpu.sync_copy(obuf, out_hbm.at[pl.ds(3 * lo, 3 * CB)])


def kernel(rgb, weights, ray_indices, num_rays):
    del num_rays
    mesh = plsc.VectorSubcoreMesh(core_axis_name="c", subcore_axis_name="s")

    acc = functools.partial(
        pl.kernel,
        out_type=jax.ShapeDtypeStruct((NC * 4 * RPAD,), jnp.float32),
        mesh=mesh,
        scratch_types=[
            pltpu.VMEM((B,), jnp.int32),
            pltpu.VMEM((B,), jnp.float32),
            pltpu.VMEM((3, B), jnp.float32),
            pltpu.VMEM((ZCH,), jnp.float32),
            pltpu.VMEM((PAGE,), jnp.int32),
            pltpu.VMEM((16,), jnp.int32),
            pltpu.VMEM((16,), jnp.float32),
            pltpu.SemaphoreType.DMA,
            pltpu.VMEM((WMAX,), jnp.float32),
            pltpu.VMEM((WMAX,), jnp.float32),
            pltpu.VMEM((WMAX,), jnp.float32),
            pltpu.VMEM((WMAX,), jnp.float32),
            pltpu.VMEM_SHARED((RPAD,), jnp.float32),
            pltpu.VMEM_SHARED((RPAD,), jnp.float32),
            pltpu.VMEM_SHARED((RPAD,), jnp.float32),
            pltpu.VMEM_SHARED((RPAD,), jnp.float32),
        ],
        compiler_params=pltpu.CompilerParams(
            needs_layout_passes=False, use_tc_tiling_on_sc=True),
        name="rgb_seg_acc",
    )(_acc_body)

    comb = functools.partial(
        pl.kernel,
        out_type=jax.ShapeDtypeStruct((OUTF,), jnp.float32),
        mesh=mesh,
        scratch_types=[pltpu.VMEM((CB,), jnp.float32) for _ in range(8)]
        + [pltpu.VMEM((3 * CB,), jnp.float32)],
        compiler_params=pltpu.CompilerParams(needs_layout_passes=False),
        name="rgb_seg_combine",
    )(_combine_body)

    part = acc(rgb.T, weights.reshape(-1), ray_indices)
    outflat = comb(part)
    return outflat[: 3 * R].reshape(R, 3)


# speedup vs baseline: 54.3597x; 1.0187x over previous
import functools

import jax
import jax.numpy as jnp
from jax import lax
from jax.experimental import pallas as pl
from jax.experimental.pallas import tpu as pltpu
from jax.experimental.pallas import tpu_sc as plsc

N = 3200000
R = 50000
RPAD = 51200
NC, NS = 2, 16
NW = NC * NS
CHUNK = 100096
B = 4096
TAIL = CHUNK - 24 * B
TAILL = N - 31 * CHUNK - 23 * B
ZCH = RPAD // NS
WMAX = 16384
PAGE = 256

CB = 1568
OUTF = 3 * NW * CB


def _acc_body(rgb_hbm, w_hbm, idx_hbm, out_hbm,
              idxv, wv, rgbv, zbuf, pidx, tiny, ecb, dsem,
              ring_r, ring_g, ring_b, ring_w,
              acc_r, acc_g, acc_b, acc_w):
    c = lax.axis_index("c")
    s = lax.axis_index("s")
    wid = s * NC + c
    it = lax.iota(jnp.int32, 16)
    itp1 = jnp.minimum(it + 1, 15)
    itm1 = jnp.maximum(it - 1, 0)
    lane0 = it == 0
    lane15 = it == 15
    zero16 = jnp.zeros((16,), jnp.float32)
    zero16i = jnp.zeros((16,), jnp.int32)
    one16i = jnp.full((16,), 1, jnp.int32)
    two16i = jnp.full((16,), 2, jnp.int32)

    def zloop(i, _):
        zbuf[pl.ds(i * 16, 16)] = zero16
        return 0
    lax.fori_loop(0, ZCH // 16, zloop, 0)
    zoff = s * ZCH
    pltpu.sync_copy(zbuf, acc_r.at[pl.ds(zoff, ZCH)])
    pltpu.sync_copy(zbuf, acc_g.at[pl.ds(zoff, ZCH)])
    pltpu.sync_copy(zbuf, acc_b.at[pl.ds(zoff, ZCH)])
    pltpu.sync_copy(zbuf, acc_w.at[pl.ds(zoff, ZCH)])

    def rz(i, _):
        d = pl.ds(i * 16, 16)
        ring_r[d] = zero16
        ring_g[d] = zero16
        ring_b[d] = zero16
        ring_w[d] = zero16
        return 0
    lax.fori_loop(0, WMAX // 16, rz, 0)
    plsc.subcore_barrier()

    chunk_base = wid * CHUNK

    pltpu.sync_copy(idx_hbm.at[pl.ds(chunk_base, 16)], tiny)
    base0 = (tiny[...][0] // PAGE) * PAGE

    def flush_page(b):
        def mkidx(k, _):
            pidx[pl.ds(k * 16, 16)] = it + (b + k * 16)
            return 0
        lax.fori_loop(0, PAGE // 16, mkidx, 0)
        pg0 = pl.multiple_of(lax.rem(b, WMAX), PAGE)
        d = pl.ds(pg0, PAGE)
        pltpu.sync_copy(ring_r.at[d], acc_r.at[pidx], add=True)
        pltpu.sync_copy(ring_g.at[d], acc_g.at[pidx], add=True)
        pltpu.sync_copy(ring_b.at[d], acc_b.at[pidx], add=True)
        pltpu.sync_copy(ring_w.at[d], acc_w.at[pidx], add=True)

        def pz(k, _):
            dd = pl.ds(pg0 + k * 16, 16)
            ring_r[dd] = zero16
            ring_g[dd] = zero16
            ring_b[dd] = zero16
            ring_w[dd] = zero16
            return 0
        lax.fori_loop(0, PAGE // 16, pz, 0)
        return b + PAGE

    def fold(prev, vcr, vcg, vcb, vcw):
        sl = jnp.full((16,), lax.rem(jnp.maximum(prev, 0), WMAX), jnp.int32)
        plsc.addupdate_scatter(ring_r, [sl], jnp.full((16,), jnp.sum(vcr), jnp.float32), mask=lane0)
        plsc.addupdate_scatter(ring_g, [sl], jnp.full((16,), jnp.sum(vcg), jnp.float32), mask=lane0)
        plsc.addupdate_scatter(ring_b, [sl], jnp.full((16,), jnp.sum(vcb), jnp.float32), mask=lane0)
        plsc.addupdate_scatter(ring_w, [sl], jnp.full((16,), jnp.sum(vcw), jnp.float32), mask=lane0)

    def process_block(boff, nsamp, carry):
        d1 = pltpu.async_copy(idx_hbm.at[pl.ds(boff, nsamp)],
                              idxv.at[pl.ds(0, nsamp)], dsem)
        d2 = pltpu.async_copy(w_hbm.at[pl.ds(boff, nsamp)],
                              wv.at[pl.ds(0, nsamp)], dsem)
        d3 = pltpu.async_copy(rgb_hbm.at[:, pl.ds(boff, nsamp)],
                              rgbv.at[:, pl.ds(0, nsamp)], dsem)
        d1.wait()
        d2.wait()
        d3.wait()

        def grp_at(off, carry):
            base, prev, vcr, vcg, vcb, vcw = carry
            idx16 = idxv[pl.ds(off, 16)]
            i0 = idx16[0]
            i15 = idx16[15]
            d16 = pl.ds(off, 16)
            wg = wv[d16]
            vr = rgbv[0, d16] * wg
            vg = rgbv[1, d16] * wg
            vb = rgbv[2, d16] * wg

            def fast(_):
                return base, prev, vcr + vr, vcg + vg, vcb + vb, vcw + wg

            def slow(_):
                fold(prev, vcr, vcg, vcb, vcw)

                def normal(_):
                    nb = lax.while_loop(
                        lambda b: i15 >= b + WMAX, flush_page, base)

                    def uni(_):
                        return nb, i15, vr, vg, vb, wg

                    def mixed(_):
                        slots = lax.rem(idx16, WMAX)
                        sh_n = plsc.load_gather(idxv, [off + itp1])
                        m = (idx16 != sh_n) | lane15
                        sh_p = plsc.load_gather(idxv, [off + itm1])
                        ms = idx16 != sh_p
                        A = plsc.cummax(jnp.where(ms, it, zero16i))
                        for ring, v in ((ring_r, vr), (ring_g, vg),
                                        (ring_b, vb), (ring_w, wg)):
                            cum = plsc.cumsum(v)
                            ecb[...] = cum - v
                            rs = plsc.load_gather(ecb, [A])
                            plsc.addupdate_scatter(ring, [slots], cum - rs, mask=m)
                        return nb, i15, zero16, zero16, zero16, zero16

                    return lax.cond(i0 == i15, uni, mixed, 0)

                def lanes(_):
                    slots = lax.rem(idx16, WMAX)
                    nb = base
                    for l in range(16):
                        ray = idx16[l]
                        nb = lax.while_loop(
                            lambda bb, ray=ray: ray >= bb + WMAX,
                            flush_page, nb)
                        lm = it == l
                        plsc.addupdate_scatter(ring_r, [slots], vr, mask=lm)
                        plsc.addupdate_scatter(ring_g, [slots], vg, mask=lm)
                        plsc.addupdate_scatter(ring_b, [slots], vb, mask=lm)
                        plsc.addupdate_scatter(ring_w, [slots], wg, mask=lm)
                    return nb, i15, zero16, zero16, zero16, zero16

                return lax.cond(i15 - i0 < WMAX - PAGE, normal, lanes, 0)

            return lax.cond((i0 == i15) & (i0 == prev), fast, slow, 0)

        def grp2(g, carry):
            carry = grp_at(g * 32, carry)
            return grp_at(g * 32 + 16, carry)

        return lax.fori_loop(0, nsamp // 32, grp2, carry)

    init = (base0, jnp.int32(-1), zero16, zero16, zero16, zero16)
    last = wid == NW - 1
    nblk = jnp.where(last, 23, 24)

    def blk_loop(blk, carry):
        return process_block(chunk_base + blk * B, B, carry)
    carry = lax.fori_loop(0, nblk, blk_loop, init)

    def tail_main(carry):
        return process_block(chunk_base + 24 * B, TAIL, carry)

    def tail_last(carry):
        return process_block(chunk_base + 23 * B, TAILL, carry)
    base, prev, vcr, vcg, vcb, vcw = lax.cond(last, tail_last, tail_main, carry)

    fold(prev, vcr, vcg, vcb, vcw)
    lax.while_loop(lambda b: prev >= b, flush_page, base)

    plsc.subcore_barrier()
    cb = c * 4 * RPAD
    pltpu.sync_copy(acc_r.at[pl.ds(zoff, ZCH)], out_hbm.at[pl.ds(cb + 0 * RPAD + zoff, ZCH)])
    pltpu.sync_copy(acc_g.at[pl.ds(zoff, ZCH)], out_hbm.at[pl.ds(cb + 1 * RPAD + zoff, ZCH)])
    pltpu.sync_copy(acc_b.at[pl.ds(zoff, ZCH)], out_hbm.at[pl.ds(cb + 2 * RPAD + zoff, ZCH)])
    pltpu.sync_copy(acc_w.at[pl.ds(zoff, ZCH)], out_hbm.at[pl.ds(cb + 3 * RPAD + zoff, ZCH)])


def _combine_body(part_hbm, out_hbm,
                  p0r, p0g, p0b, p0w, p1r, p1g, p1b, p1w, obuf):
    c = lax.axis_index("c")
    s = lax.axis_index("s")
    wid = s * NC + c
    it = lax.iota(jnp.int32, 16)
    it3 = it * 3
    bufs = (p0r, p0g, p0b, p0w, p1r, p1g, p1b, p1w)
    lo = wid * CB
    for k, b in enumerate(bufs):
        pltpu.sync_copy(part_hbm.at[pl.ds(k * RPAD + lo, CB)], b)

    def grp(g, _):
        off = g * 16
        d = pl.ds(off, 16)
        aw = p0w[d] + p1w[d]
        bg = 1.0 - aw
        orr = p0r[d] + p1r[d] + bg
        ogg = p0g[d] + p1g[d] + bg
        obb = p0b[d] + p1b[d] + bg
        pos = it3 + off * 3
        plsc.store_scatter(obuf, [pos], orr)
        plsc.store_scatter(obuf, [pos + 1], ogg)
        plsc.store_scatter(obuf, [pos + 2], obb)
        return 0
    lax.fori_loop(0, CB // 16, grp, 0)
    pltpu.sync_copy(obuf, out_hbm.at[pl.ds(3 * lo, 3 * CB)])


def kernel(rgb, weights, ray_indices, num_rays):
    del num_rays
    mesh = plsc.VectorSubcoreMesh(core_axis_name="c", subcore_axis_name="s")

    acc = functools.partial(
        pl.kernel,
        out_type=jax.ShapeDtypeStruct((NC * 4 * RPAD,), jnp.float32),
        mesh=mesh,
        scratch_types=[
            pltpu.VMEM((B,), jnp.int32),
            pltpu.VMEM((B,), jnp.float32),
            pltpu.VMEM((3, B), jnp.float32),
            pltpu.VMEM((ZCH,), jnp.float32),
            pltpu.VMEM((PAGE,), jnp.int32),
            pltpu.VMEM((16,), jnp.int32),
            pltpu.VMEM((16,), jnp.float32),
            pltpu.SemaphoreType.DMA,
            pltpu.VMEM((WMAX,), jnp.float32),
            pltpu.VMEM((WMAX,), jnp.float32),
            pltpu.VMEM((WMAX,), jnp.float32),
            pltpu.VMEM((WMAX,), jnp.float32),
            pltpu.VMEM_SHARED((RPAD,), jnp.float32),
            pltpu.VMEM_SHARED((RPAD,), jnp.float32),
            pltpu.VMEM_SHARED((RPAD,), jnp.float32),
            pltpu.VMEM_SHARED((RPAD,), jnp.float32),
        ],
        compiler_params=pltpu.CompilerParams(
            needs_layout_passes=False, use_tc_tiling_on_sc=True),
        name="rgb_seg_acc",
    )(_acc_body)

    comb = functools.partial(
        pl.kernel,
        out_type=jax.ShapeDtypeStruct((OUTF,), jnp.float32),
        mesh=mesh,
        scratch_types=[pltpu.VMEM((CB,), jnp.float32) for _ in range(8)]
        + [pltpu.VMEM((3 * CB,), jnp.float32)],
        compiler_params=pltpu.CompilerParams(needs_layout_passes=False),
        name="rgb_seg_combine",
    )(_combine_body)

    part = acc(rgb.T, weights.reshape(-1), ray_indices)
    outflat = comb(part)
    return outflat[: 3 * R].reshape(R, 3)
